# Initial kernel scaffold; baseline (speedup 1.0000x reference)
import jax
import jax.numpy as jnp
from jax.experimental import pallas as pl


def kernel(x, edge_index, edge_attr, eps, params):
    raise NotImplementedError("write your pallas kernel here")


# speedup vs baseline: 2.1506x; 2.1506x over previous
import functools

import jax
import jax.numpy as jnp
from jax import lax
from jax.experimental import pallas as pl
from jax.experimental.pallas import tpu as pltpu
from jax.experimental.pallas import tpu_sc as plsc

N = 10000
E = 320000
D_IN = 128
D_EDGE = 16
H = 32
LAT = 32

N_PAD = 10240
E_PAD = 323584
CHUNK = 128
NW = 32
TILE_ROWS = N_PAD // 16

BLK_N = 1280
BLK_E = 4096
NBLK_E = E_PAD // BLK_E

_F32 = jnp.float32


def _leaky(v):
    return jax.nn.leaky_relu(v, 0.01)


def _dot(a, b):
    return jnp.dot(a, b, preferred_element_type=_F32)


def _sc_mesh():
    return plsc.VectorSubcoreMesh(core_axis_name="c", subcore_axis_name="s")


_SC_PARAMS = pltpu.CompilerParams(use_tc_tiling_on_sc=False)


def _sc_gather(h_pad, idx_all):
    tot = idx_all.shape[0]
    per_w = tot // NW
    nchunk = per_w // CHUNK

    @functools.partial(
        pl.kernel,
        mesh=_sc_mesh(),
        compiler_params=_SC_PARAMS,
        out_type=jax.ShapeDtypeStruct((tot, H), _F32),
        scratch_types=[
            pltpu.VMEM((CHUNK,), jnp.int32),
            pltpu.VMEM((CHUNK, H), _F32),
            pltpu.SemaphoreType.DMA,
        ],
    )
    def k(h_hbm, idx_hbm, out_hbm, idx_v, rows_v, sem):
        wid = lax.axis_index("s") * 2 + lax.axis_index("c")

        @pl.loop(0, nchunk)
        def _(j):
            base = wid * per_w + j * CHUNK
            pltpu.sync_copy(idx_hbm.at[pl.ds(base, CHUNK)], idx_v)
            pltpu.async_copy(h_hbm.at[idx_v], rows_v, sem).wait()
            pltpu.sync_copy(rows_v, out_hbm.at[pl.ds(base, CHUNK)])

    return k(h_pad, idx_all)


def _sc_segsum(m, dst_idx, zeros_nh):
    per_w = E_PAD // NW
    nchunk = per_w // CHUNK

    @functools.partial(
        pl.kernel,
        mesh=_sc_mesh(),
        compiler_params=_SC_PARAMS,
        out_type=jax.ShapeDtypeStruct((2, N_PAD, H), _F32),
        scratch_types=[
            pltpu.VMEM((CHUNK,), jnp.int32),
            pltpu.VMEM((CHUNK, H), _F32),
            pltpu.VMEM_SHARED((N_PAD, H), _F32),
        ],
    )
    def k(m_hbm, dst_hbm, z_hbm, out_hbm, idx_v, rows_v, acc):
        cid = lax.axis_index("c")
        sid = lax.axis_index("s")
        wid = sid * 2 + cid
        row0 = sid * TILE_ROWS
        pltpu.sync_copy(z_hbm.at[pl.ds(row0, TILE_ROWS)],
                        acc.at[pl.ds(row0, TILE_ROWS)])
        plsc.subcore_barrier()

        @pl.loop(0, nchunk)
        def _(j):
            base = wid * per_w + j * CHUNK
            pltpu.sync_copy(dst_hbm.at[pl.ds(base, CHUNK)], idx_v)
            pltpu.sync_copy(m_hbm.at[pl.ds(base, CHUNK)], rows_v)
            pltpu.sync_copy(rows_v, acc.at[idx_v], add=True)

        plsc.subcore_barrier()
        pltpu.sync_copy(acc.at[pl.ds(row0, TILE_ROWS)],
                        out_hbm.at[cid, pl.ds(row0, TILE_ROWS)])

    return k(m, dst_idx, zeros_nh)


def _sc_hist(dst_idx, zeros_n16, ones_c16):
    per_w = E_PAD // NW
    nchunk = per_w // CHUNK

    @functools.partial(
        pl.kernel,
        mesh=_sc_mesh(),
        compiler_params=_SC_PARAMS,
        out_type=jax.ShapeDtypeStruct((2, N_PAD, 16), _F32),
        scratch_types=[
            pltpu.VMEM((CHUNK,), jnp.int32),
            pltpu.VMEM((CHUNK, 16), _F32),
            pltpu.VMEM_SHARED((N_PAD, 16), _F32),
        ],
    )
    def k(dst_hbm, z_hbm, ones_hbm, out_hbm, idx_v, ones_v, acc):
        cid = lax.axis_index("c")
        sid = lax.axis_index("s")
        wid = sid * 2 + cid
        row0 = sid * TILE_ROWS
        pltpu.sync_copy(ones_hbm, ones_v)
        pltpu.sync_copy(z_hbm.at[pl.ds(row0, TILE_ROWS)],
                        acc.at[pl.ds(row0, TILE_ROWS)])
        plsc.subcore_barrier()

        @pl.loop(0, nchunk)
        def _(j):
            base = wid * per_w + j * CHUNK
            pltpu.sync_copy(dst_hbm.at[pl.ds(base, CHUNK)], idx_v)
            pltpu.sync_copy(ones_v, acc.at[idx_v], add=True)

        plsc.subcore_barrier()
        pltpu.sync_copy(acc.at[pl.ds(row0, TILE_ROWS)],
                        out_hbm.at[cid, pl.ds(row0, TILE_ROWS)])

    return k(dst_idx, zeros_n16, ones_c16)


def _tc_embed(x_pad, w_in, b_in):
    def body(x_ref, w_ref, b_ref, o_ref):
        o_ref[...] = _leaky(_dot(x_ref[...], w_ref[...]) + b_ref[...])

    return pl.pallas_call(
        body,
        grid=(N_PAD // BLK_N,),
        in_specs=[
            pl.BlockSpec((BLK_N, D_IN), lambda i: (i, 0)),
            pl.BlockSpec((D_IN, H), lambda i: (0, 0)),
            pl.BlockSpec((1, H), lambda i: (0, 0)),
        ],
        out_specs=pl.BlockSpec((BLK_N, H), lambda i: (i, 0)),
        out_shape=jax.ShapeDtypeStruct((N_PAD, H), _F32),
    )(x_pad, w_in, b_in)


def _tc_edge(g_all, ea_pad, w_e, b_e, w_d, w_s, w_ee, b_cat):
    def body(hd_ref, hs_ref, ea_ref, we_ref, be_ref, wd_ref, ws_ref,
             wee_ref, bc_ref, m_ref):
        e = _leaky(_dot(ea_ref[...], we_ref[...]) + be_ref[...])
        pre = (_dot(hd_ref[...], wd_ref[...])
               + _dot(hs_ref[...], ws_ref[...])
               + _dot(e, wee_ref[...])
               + bc_ref[...])
        m_ref[...] = jax.nn.sigmoid(pre[:, :H]) * jax.nn.softplus(pre[:, H:])

    return pl.pallas_call(
        body,
        grid=(NBLK_E,),
        in_specs=[
            pl.BlockSpec((BLK_E, H), lambda i: (i, 0)),
            pl.BlockSpec((BLK_E, H), lambda i: (i + NBLK_E, 0)),
            pl.BlockSpec((BLK_E, D_EDGE), lambda i: (i, 0)),
            pl.BlockSpec((D_EDGE, H), lambda i: (0, 0)),
            pl.BlockSpec((1, H), lambda i: (0, 0)),
            pl.BlockSpec((H, 2 * H), lambda i: (0, 0)),
            pl.BlockSpec((H, 2 * H), lambda i: (0, 0)),
            pl.BlockSpec((H, 2 * H), lambda i: (0, 0)),
            pl.BlockSpec((1, 2 * H), lambda i: (0, 0)),
        ],
        out_specs=pl.BlockSpec((BLK_E, H), lambda i: (i, 0)),
        out_shape=jax.ShapeDtypeStruct((E_PAD, H), _F32),
    )(g_all, g_all, ea_pad, w_e, b_e, w_d, w_s, w_ee, b_cat)


def _tc_update(h, aggp, cntp):
    def body(h_ref, p0_ref, p1_ref, c0_ref, c1_ref, o_ref):
        cnt = jnp.maximum(c0_ref[0, :, 0:1] + c1_ref[0, :, 0:1], 1.0)
        o_ref[...] = h_ref[...] + (p0_ref[0] + p1_ref[0]) / cnt

    return pl.pallas_call(
        body,
        grid=(N_PAD // BLK_N,),
        in_specs=[
            pl.BlockSpec((BLK_N, H), lambda i: (i, 0)),
            pl.BlockSpec((1, BLK_N, H), lambda i: (0, i, 0)),
            pl.BlockSpec((1, BLK_N, H), lambda i: (1, i, 0)),
            pl.BlockSpec((1, BLK_N, 16), lambda i: (0, i, 0)),
            pl.BlockSpec((1, BLK_N, 16), lambda i: (1, i, 0)),
        ],
        out_specs=pl.BlockSpec((BLK_N, H), lambda i: (i, 0)),
        out_shape=jax.ShapeDtypeStruct((N_PAD, H), _F32),
    )(h, aggp, aggp, cntp, cntp)


def _tc_final(h, aggp, cntp, eps_pad, w_enc, b_enc, w_d0, b_d0, w_d1, b_d1,
              w_d2, b_d2, w_do, b_do):
    def body(h_ref, p0_ref, p1_ref, c0_ref, c1_ref, eps_ref,
             wenc_ref, benc_ref, wd0_ref, bd0_ref, wd1_ref, bd1_ref,
             wd2_ref, bd2_ref, wdo_ref, bdo_ref,
             recon_ref, mu_ref, lv_ref):
        cnt = jnp.maximum(c0_ref[0, :, 0:1] + c1_ref[0, :, 0:1], 1.0)
        h2 = h_ref[...] + (p0_ref[0] + p1_ref[0]) / cnt
        enc = _leaky(_dot(h2, wenc_ref[...]) + benc_ref[...])
        mu = enc[:, :LAT]
        lv = enc[:, LAT:]
        zl = mu + jnp.exp(0.5 * lv) * eps_ref[...]
        d = _leaky(_dot(zl, wd0_ref[...]) + bd0_ref[...])
        d = _leaky(_dot(d, wd1_ref[...]) + bd1_ref[...])
        d = _leaky(_dot(d, wd2_ref[...]) + bd2_ref[...])
        recon_ref[...] = jax.nn.sigmoid(_dot(d, wdo_ref[...]) + bdo_ref[...])
        mu_ref[...] = mu
        lv_ref[...] = lv

    full = lambda shape: pl.BlockSpec(shape, lambda i: tuple(0 for _ in shape))
    return pl.pallas_call(
        body,
        grid=(N_PAD // BLK_N,),
        in_specs=[
            pl.BlockSpec((BLK_N, H), lambda i: (i, 0)),
            pl.BlockSpec((1, BLK_N, H), lambda i: (0, i, 0)),
            pl.BlockSpec((1, BLK_N, H), lambda i: (1, i, 0)),
            pl.BlockSpec((1, BLK_N, 16), lambda i: (0, i, 0)),
            pl.BlockSpec((1, BLK_N, 16), lambda i: (1, i, 0)),
            pl.BlockSpec((BLK_N, LAT), lambda i: (i, 0)),
            full((H, 2 * LAT)), full((1, 2 * LAT)),
            full((LAT, H)), full((1, H)),
            full((H, H)), full((1, H)),
            full((H, H)), full((1, H)),
            full((H, D_IN)), full((1, D_IN)),
        ],
        out_specs=[
            pl.BlockSpec((BLK_N, D_IN), lambda i: (i, 0)),
            pl.BlockSpec((BLK_N, LAT), lambda i: (i, 0)),
            pl.BlockSpec((BLK_N, LAT), lambda i: (i, 0)),
        ],
        out_shape=[
            jax.ShapeDtypeStruct((N_PAD, D_IN), _F32),
            jax.ShapeDtypeStruct((N_PAD, LAT), _F32),
            jax.ShapeDtypeStruct((N_PAD, LAT), _F32),
        ],
    )(h, aggp, aggp, cntp, cntp, eps_pad,
      w_enc, b_enc, w_d0, b_d0, w_d1, b_d1, w_d2, b_d2, w_do, b_do)


def kernel(x, edge_index, edge_attr, eps, params):
    ei = edge_index.astype(jnp.int32)
    pad_idx = jnp.full((E_PAD - E,), N, jnp.int32)
    src = jnp.concatenate([ei[0], pad_idx])
    dst = jnp.concatenate([ei[1], pad_idx])
    idx_all = jnp.concatenate([dst, src])

    x_pad = jnp.pad(x, ((0, N_PAD - N), (0, 0)))
    ea_pad = jnp.pad(edge_attr, ((0, E_PAD - E), (0, 0)))
    eps_pad = jnp.pad(eps, ((0, N_PAD - N), (0, 0)))
    zeros_nh = jnp.zeros((N_PAD, H), _F32)
    zeros_n16 = jnp.zeros((N_PAD, 16), _F32)
    ones_c16 = jnp.ones((CHUNK, 16), _F32)

    p = params
    b2 = lambda b: b.reshape(1, -1).astype(_F32)

    cntp = _sc_hist(dst, zeros_n16, ones_c16)
    h = _tc_embed(x_pad, p['W_in'], b2(p['b_in']))

    for i in range(2):
        wf, ws = p['Wf'][i], p['Ws'][i]
        w_d = jnp.concatenate([wf[:H], ws[:H]], axis=1)
        w_s = jnp.concatenate([wf[H:2 * H], ws[H:2 * H]], axis=1)
        w_ee = jnp.concatenate([wf[2 * H:], ws[2 * H:]], axis=1)
        b_cat = jnp.concatenate([p['bf'][i], p['bs'][i]]).reshape(1, -1)

        g_all = _sc_gather(h, idx_all)
        m = _tc_edge(g_all, ea_pad, p['W_e'], b2(p['b_e']), w_d, w_s, w_ee,
                     b_cat)
        aggp = _sc_segsum(m, dst, zeros_nh)
        if i == 0:
            h = _tc_update(h, aggp, cntp)

    recon, mu, lv = _tc_final(
        h, aggp, cntp, eps_pad,
        p['W_enc'], b2(p['b_enc']), p['W_d0'], b2(p['b_d0']),
        p['W_d1'], b2(p['b_d1']), p['W_d2'], b2(p['b_d2']),
        p['W_dout'], b2(p['b_dout']))
    return recon[:N], mu[:N], lv[:N]


# speedup vs baseline: 3.7030x; 1.7218x over previous
import functools

import jax
import jax.numpy as jnp
from jax import lax
from jax.experimental import pallas as pl
from jax.experimental.pallas import tpu as pltpu
from jax.experimental.pallas import tpu_sc as plsc

N = 10000
E = 320000
D_IN = 128
D_EDGE = 16
H = 32
LAT = 32

N_PAD = 10240
E_PAD = 327680
TOT = 2 * E_PAD
CH = 128
P = 8
NROW = CH // P
WIDE = P * H
NBUF = 8
NW = 32
TILE_ROWS = N_PAD // 16

BLK_N = 1280
BLK_E8 = 512
NBLK_E8 = (E_PAD // P) // BLK_E8

_F32 = jnp.float32


def _leaky(v):
    return jax.nn.leaky_relu(v, 0.01)


def _dot(a, b):
    return jnp.dot(a, b, preferred_element_type=_F32)


def _sc_mesh():
    return plsc.VectorSubcoreMesh(core_axis_name="c", subcore_axis_name="s")


_SC_PARAMS = pltpu.CompilerParams(use_tc_tiling_on_sc=False)


def _repack_32_to_256(src, dst, b):
    @pl.loop(0, NROW)
    def _(tt):
        for k in range(16):
            dst[b, tt, pl.ds(k * 16, 16)] = src[b, tt * 8 + k // 2,
                                                pl.ds((k % 2) * 16, 16)]


def _repack_256_to_32(src, dst, b):
    @pl.loop(0, NROW)
    def _(tt):
        for k in range(16):
            dst[b, tt * 8 + k // 2, pl.ds((k % 2) * 16, 16)] = src[
                b, tt, pl.ds(k * 16, 16)]


def _sc_gather(h_pad, idx2):
    nround = (TOT // NW) // (CH * NBUF)
    per_w_rows = nround * NBUF

    @functools.partial(
        pl.kernel,
        mesh=_sc_mesh(),
        compiler_params=_SC_PARAMS,
        out_type=jax.ShapeDtypeStruct((TOT // P, WIDE), _F32),
        scratch_types=[
            pltpu.VMEM((NBUF, CH), jnp.int32),
            pltpu.VMEM((NBUF, CH, H), _F32),
            pltpu.VMEM((NBUF, NROW, WIDE), _F32),
            pltpu.SemaphoreType.DMA((NBUF,)),
            pltpu.SemaphoreType.DMA((NBUF,)),
        ],
    )
    def k(h_hbm, i_hbm, o_hbm, iv, rv32, rv256, semg, semw):
        wid = lax.axis_index("s") * 2 + lax.axis_index("c")
        row0 = wid * per_w_rows

        def round_body(r, first):
            pltpu.sync_copy(i_hbm.at[pl.ds(row0 + r * NBUF, NBUF)], iv)
            for b in range(NBUF):
                if not first:
                    pltpu.make_async_copy(rv256.at[b],
                                          o_hbm.at[pl.ds(0, NROW)],
                                          semw.at[b]).wait()
                pltpu.async_copy(h_hbm.at[iv.at[b]], rv32.at[b], semg.at[b])
            for b in range(NBUF):
                pltpu.make_async_copy(h_hbm.at[iv.at[b]], rv32.at[b],
                                      semg.at[b]).wait()
                _repack_32_to_256(rv32, rv256, b)
                obase = (row0 + r * NBUF + b) * NROW
                pltpu.async_copy(rv256.at[b], o_hbm.at[pl.ds(obase, NROW)],
                                 semw.at[b])

        round_body(0, True)

        @pl.loop(1, nround)
        def _(r):
            round_body(r, False)

        for b in range(NBUF):
            pltpu.make_async_copy(rv256.at[b], o_hbm.at[pl.ds(0, NROW)],
                                  semw.at[b]).wait()

    return k(h_pad, idx2)


def _sc_segsum(m8, dst2, zeros_nh):
    nround = (E_PAD // NW) // (CH * NBUF)
    per_w_rows = nround * NBUF

    @functools.partial(
        pl.kernel,
        mesh=_sc_mesh(),
        compiler_params=_SC_PARAMS,
        out_type=jax.ShapeDtypeStruct((2, N_PAD, H), _F32),
        scratch_types=[
            pltpu.VMEM((NBUF, CH), jnp.int32),
            pltpu.VMEM((NBUF, NROW, WIDE), _F32),
            pltpu.VMEM((NBUF, CH, H), _F32),
            pltpu.VMEM_SHARED((N_PAD, H), _F32),
            pltpu.SemaphoreType.DMA((NBUF,)),
            pltpu.SemaphoreType.DMA((NBUF,)),
        ],
    )
    def k(m_hbm, i_hbm, z_hbm, o_hbm, iv, rv256, rv32, acc, seml, sems):
        cid = lax.axis_index("c")
        sid = lax.axis_index("s")
        wid = sid * 2 + cid
        row0 = wid * per_w_rows
        r0 = sid * TILE_ROWS
        pltpu.sync_copy(z_hbm.at[pl.ds(r0, TILE_ROWS)],
                        acc.at[pl.ds(r0, TILE_ROWS)])
        plsc.subcore_barrier()

        @pl.loop(0, nround)
        def _(r):
            pltpu.sync_copy(i_hbm.at[pl.ds(row0 + r * NBUF, NBUF)], iv)
            for b in range(NBUF):
                mbase = (row0 + r * NBUF + b) * NROW
                pltpu.async_copy(m_hbm.at[pl.ds(mbase, NROW)], rv256.at[b],
                                 seml.at[b])
            for b in range(NBUF):
                pltpu.make_async_copy(m_hbm.at[pl.ds(0, NROW)], rv256.at[b],
                                      seml.at[b]).wait()
                _repack_256_to_32(rv256, rv32, b)
                pltpu.async_copy(rv32.at[b], acc.at[iv.at[b]], sems.at[b],
                                 add=True)
            for b in range(NBUF):
                pltpu.make_async_copy(rv32.at[b], acc.at[iv.at[b]],
                                      sems.at[b]).wait()

        plsc.subcore_barrier()
        pltpu.sync_copy(acc.at[pl.ds(r0, TILE_ROWS)],
                        o_hbm.at[cid, pl.ds(r0, TILE_ROWS)])

    return k(m8, dst2, zeros_nh)


def _sc_hist(dst2, zeros_n16, ones_c16):
    per_w_rows = (E_PAD // NW) // CH

    @functools.partial(
        pl.kernel,
        mesh=_sc_mesh(),
        compiler_params=_SC_PARAMS,
        out_type=jax.ShapeDtypeStruct((2, N_PAD, 16), _F32),
        scratch_types=[
            pltpu.VMEM((per_w_rows, CH), jnp.int32),
            pltpu.VMEM((CH, 16), _F32),
            pltpu.VMEM_SHARED((N_PAD, 16), _F32),
            pltpu.SemaphoreType.DMA,
        ],
    )
    def k(i_hbm, z_hbm, ones_hbm, o_hbm, iv, ones_v, acc, sem):
        cid = lax.axis_index("c")
        sid = lax.axis_index("s")
        wid = sid * 2 + cid
        row0 = wid * per_w_rows
        r0 = sid * TILE_ROWS
        pltpu.sync_copy(ones_hbm, ones_v)
        pltpu.sync_copy(i_hbm.at[pl.ds(row0, per_w_rows)], iv)
        pltpu.sync_copy(z_hbm.at[pl.ds(r0, TILE_ROWS)],
                        acc.at[pl.ds(r0, TILE_ROWS)])
        plsc.subcore_barrier()

        @pl.loop(0, per_w_rows // NBUF)
        def _(r):
            for b in range(NBUF):
                pltpu.async_copy(ones_v, acc.at[iv.at[r * NBUF + b]], sem,
                                 add=True)
            for b in range(NBUF):
                pltpu.make_async_copy(ones_v, acc.at[iv.at[r * NBUF + b]],
                                      sem).wait()

        plsc.subcore_barrier()
        pltpu.sync_copy(acc.at[pl.ds(r0, TILE_ROWS)],
                        o_hbm.at[cid, pl.ds(r0, TILE_ROWS)])

    return k(dst2, zeros_n16, ones_c16)


def _tc_embed(x_pad, w_in, b_in):
    def body(x_ref, w_ref, b_ref, o_ref):
        o_ref[...] = _leaky(_dot(x_ref[...], w_ref[...]) + b_ref[...])

    return pl.pallas_call(
        body,
        grid=(N_PAD // BLK_N,),
        in_specs=[
            pl.BlockSpec((BLK_N, D_IN), lambda i: (i, 0)),
            pl.BlockSpec((D_IN, H), lambda i: (0, 0)),
            pl.BlockSpec((1, H), lambda i: (0, 0)),
        ],
        out_specs=pl.BlockSpec((BLK_N, H), lambda i: (i, 0)),
        out_shape=jax.ShapeDtypeStruct((N_PAD, H), _F32),
    )(x_pad, w_in, b_in)


def _tc_edge(g8, ea8, w_e8, b_e8, w_d, w_s, w_ee, b_cat8):
    def body(hd_ref, hs_ref, ea_ref, we_ref, be_ref, wd_ref, ws_ref,
             wee_ref, bc_ref, m_ref):
        e8 = _leaky(_dot(ea_ref[...], we_ref[...]) + be_ref[...])
        pre = (_dot(hd_ref[...], wd_ref[...])
               + _dot(hs_ref[...], ws_ref[...])
               + _dot(e8, wee_ref[...])
               + bc_ref[...])
        m_ref[...] = jax.nn.sigmoid(pre[:, :WIDE]) * jax.nn.softplus(
            pre[:, WIDE:])

    return pl.pallas_call(
        body,
        grid=(NBLK_E8,),
        in_specs=[
            pl.BlockSpec((BLK_E8, WIDE), lambda i: (i, 0)),
            pl.BlockSpec((BLK_E8, WIDE), lambda i: (i + NBLK_E8, 0)),
            pl.BlockSpec((BLK_E8, 128), lambda i: (i, 0)),
            pl.BlockSpec((128, WIDE), lambda i: (0, 0)),
            pl.BlockSpec((1, WIDE), lambda i: (0, 0)),
            pl.BlockSpec((WIDE, 2 * WIDE), lambda i: (0, 0)),
            pl.BlockSpec((WIDE, 2 * WIDE), lambda i: (0, 0)),
            pl.BlockSpec((WIDE, 2 * WIDE), lambda i: (0, 0)),
            pl.BlockSpec((1, 2 * WIDE), lambda i: (0, 0)),
        ],
        out_specs=pl.BlockSpec((BLK_E8, WIDE), lambda i: (i, 0)),
        out_shape=jax.ShapeDtypeStruct((E_PAD // P, WIDE), _F32),
    )(g8, g8, ea8, w_e8, b_e8, w_d, w_s, w_ee, b_cat8)


def _tc_update(h, aggp, cntp):
    def body(h_ref, p0_ref, p1_ref, c0_ref, c1_ref, o_ref):
        cnt = jnp.maximum(c0_ref[0, :, 0:1] + c1_ref[0, :, 0:1], 1.0)
        o_ref[...] = h_ref[...] + (p0_ref[0] + p1_ref[0]) / cnt

    return pl.pallas_call(
        body,
        grid=(N_PAD // BLK_N,),
        in_specs=[
            pl.BlockSpec((BLK_N, H), lambda i: (i, 0)),
            pl.BlockSpec((1, BLK_N, H), lambda i: (0, i, 0)),
            pl.BlockSpec((1, BLK_N, H), lambda i: (1, i, 0)),
            pl.BlockSpec((1, BLK_N, 16), lambda i: (0, i, 0)),
            pl.BlockSpec((1, BLK_N, 16), lambda i: (1, i, 0)),
        ],
        out_specs=pl.BlockSpec((BLK_N, H), lambda i: (i, 0)),
        out_shape=jax.ShapeDtypeStruct((N_PAD, H), _F32),
    )(h, aggp, aggp, cntp, cntp)


def _tc_final(h, aggp, cntp, eps_pad, w_enc, b_enc, w_d0, b_d0, w_d1, b_d1,
              w_d2, b_d2, w_do, b_do):
    def body(h_ref, p0_ref, p1_ref, c0_ref, c1_ref, eps_ref,
             wenc_ref, benc_ref, wd0_ref, bd0_ref, wd1_ref, bd1_ref,
             wd2_ref, bd2_ref, wdo_ref, bdo_ref,
             recon_ref, mu_ref, lv_ref):
        cnt = jnp.maximum(c0_ref[0, :, 0:1] + c1_ref[0, :, 0:1], 1.0)
        h2 = h_ref[...] + (p0_ref[0] + p1_ref[0]) / cnt
        enc = _leaky(_dot(h2, wenc_ref[...]) + benc_ref[...])
        mu = enc[:, :LAT]
        lv = enc[:, LAT:]
        zl = mu + jnp.exp(0.5 * lv) * eps_ref[...]
        d = _leaky(_dot(zl, wd0_ref[...]) + bd0_ref[...])
        d = _leaky(_dot(d, wd1_ref[...]) + bd1_ref[...])
        d = _leaky(_dot(d, wd2_ref[...]) + bd2_ref[...])
        recon_ref[...] = jax.nn.sigmoid(_dot(d, wdo_ref[...]) + bdo_ref[...])
        mu_ref[...] = mu
        lv_ref[...] = lv

    full = lambda shape: pl.BlockSpec(shape, lambda i: tuple(0 for _ in shape))
    return pl.pallas_call(
        body,
        grid=(N_PAD // BLK_N,),
        in_specs=[
            pl.BlockSpec((BLK_N, H), lambda i: (i, 0)),
            pl.BlockSpec((1, BLK_N, H), lambda i: (0, i, 0)),
            pl.BlockSpec((1, BLK_N, H), lambda i: (1, i, 0)),
            pl.BlockSpec((1, BLK_N, 16), lambda i: (0, i, 0)),
            pl.BlockSpec((1, BLK_N, 16), lambda i: (1, i, 0)),
            pl.BlockSpec((BLK_N, LAT), lambda i: (i, 0)),
            full((H, 2 * LAT)), full((1, 2 * LAT)),
            full((LAT, H)), full((1, H)),
            full((H, H)), full((1, H)),
            full((H, H)), full((1, H)),
            full((H, D_IN)), full((1, D_IN)),
        ],
        out_specs=[
            pl.BlockSpec((BLK_N, D_IN), lambda i: (i, 0)),
            pl.BlockSpec((BLK_N, LAT), lambda i: (i, 0)),
            pl.BlockSpec((BLK_N, LAT), lambda i: (i, 0)),
        ],
        out_shape=[
            jax.ShapeDtypeStruct((N_PAD, D_IN), _F32),
            jax.ShapeDtypeStruct((N_PAD, LAT), _F32),
            jax.ShapeDtypeStruct((N_PAD, LAT), _F32),
        ],
    )(h, aggp, aggp, cntp, cntp, eps_pad,
      w_enc, b_enc, w_d0, b_d0, w_d1, b_d1, w_d2, b_d2, w_do, b_do)


def kernel(x, edge_index, edge_attr, eps, params):
    ei = edge_index.astype(jnp.int32)
    pad_idx = jnp.full((E_PAD - E,), N, jnp.int32)
    src = jnp.concatenate([ei[0], pad_idx])
    dst = jnp.concatenate([ei[1], pad_idx])
    idx2 = jnp.concatenate([dst, src]).reshape(TOT // CH, CH)
    dst2 = dst.reshape(E_PAD // CH, CH)

    x_pad = jnp.pad(x, ((0, N_PAD - N), (0, 0)))
    ea8 = jnp.pad(edge_attr.reshape(E // P, P * D_EDGE),
                  ((0, (E_PAD - E) // P), (0, 0)))
    eps_pad = jnp.pad(eps, ((0, N_PAD - N), (0, 0)))
    zeros_nh = jnp.zeros((N_PAD, H), _F32)
    zeros_n16 = jnp.zeros((N_PAD, 16), _F32)
    ones_c16 = jnp.ones((CH, 16), _F32)

    p = params
    b2 = lambda b: b.reshape(1, -1).astype(_F32)
    i8 = jnp.eye(P, dtype=_F32)
    kron = lambda w: jnp.kron(i8, w)

    w_e8 = kron(p['W_e'])
    b_e8 = jnp.tile(p['b_e'], P).reshape(1, WIDE)

    cntp = _sc_hist(dst2, zeros_n16, ones_c16)
    h = _tc_embed(x_pad, p['W_in'], b2(p['b_in']))

    for i in range(2):
        wf, ws = p['Wf'][i], p['Ws'][i]
        w_d = jnp.concatenate([kron(wf[:H]), kron(ws[:H])], axis=1)
        w_s = jnp.concatenate([kron(wf[H:2 * H]), kron(ws[H:2 * H])], axis=1)
        w_ee = jnp.concatenate([kron(wf[2 * H:]), kron(ws[2 * H:])], axis=1)
        b_cat8 = jnp.concatenate([jnp.tile(p['bf'][i], P),
                                  jnp.tile(p['bs'][i], P)]).reshape(1, 2 * WIDE)

        g8 = _sc_gather(h, idx2)
        m8 = _tc_edge(g8, ea8, w_e8, b_e8, w_d, w_s, w_ee, b_cat8)
        aggp = _sc_segsum(m8, dst2, zeros_nh)
        if i == 0:
            h = _tc_update(h, aggp, cntp)

    recon, mu, lv = _tc_final(
        h, aggp, cntp, eps_pad,
        p['W_enc'], b2(p['b_enc']), p['W_d0'], b2(p['b_d0']),
        p['W_d1'], b2(p['b_d1']), p['W_d2'], b2(p['b_d2']),
        p['W_dout'], b2(p['b_dout']))
    return recon[:N], mu[:N], lv[:N]
